# Initial kernel scaffold; baseline (speedup 1.0000x reference)
import jax
import jax.numpy as jnp
from jax.experimental import pallas as pl


def kernel(node_types, node_labels, edge_types, edge_labels, edge_index, emb, W_node0, W_src0, W_edge0, b0, W_node1, W_src1, W_edge1, b1, fc_w, fc_b):
    raise NotImplementedError("write your pallas kernel here")


# speedup vs baseline: 7.0594x; 7.0594x over previous
import jax
import jax.numpy as jnp
from jax import lax
from jax.experimental import pallas as pl
from jax.experimental.pallas import tpu as pltpu
from jax.experimental.pallas import tpu_sc as plsc

N = 100000
E = 1600000
NUM_TOK = 10000
H = 16

NC = 2
NS = 16
L = 16
NW = NC * NS

CHUNK = 128
N_FULL = N // CHUNK
N_TAIL = N - N_FULL * CHUNK
NODE_ITERS = (N_FULL + NW - 1) // NW

E_CHUNKS = E // CHUNK
E_ITERS = (E_CHUNKS + NW - 1) // NW

NPS = 6272
N_ACC = NS * NPS

_mesh = lambda: plsc.VectorSubcoreMesh(core_axis_name="c", subcore_axis_name="s")


def _iota():
    return lax.iota(jnp.int32, L)


def _build_t0_body(nt_hbm, lab_hbm, emb_hbm, t0_hbm, emb_v, nt_v, lab_v, out_v):
    c = lax.axis_index("c")
    s = lax.axis_index("s")
    w = s * NC + c
    pltpu.sync_copy(emb_hbm, emb_v)
    it = _iota()
    six = jnp.full((L,), 6, jnp.int32)
    eight = jnp.full((L,), 8, jnp.int32)

    def do_chunk(base, k):
        pltpu.sync_copy(nt_hbm.at[pl.ds(base * 6, k * 6)], nt_v.at[pl.ds(0, k * 6)])
        pltpu.sync_copy(lab_hbm.at[pl.ds(base, k)], lab_v.at[pl.ds(0, k)])
        for j in range((k * 6) // L):
            vals = nt_v[pl.ds(j * L, L)]
            p = j * L + it
            tgt = lax.div(p, six) * eight + lax.rem(p, six)
            plsc.store_scatter(out_v, [tgt], vals)
        for g in range(k // L):
            labs = lab_v[pl.ds(g * L, L)]
            e = plsc.load_gather(emb_v, [labs])
            node = g * L + it
            plsc.store_scatter(out_v, [node * eight + 6], e)
            plsc.store_scatter(out_v, [node * eight + 7],
                               jnp.zeros((L,), jnp.float32))
        pltpu.sync_copy(out_v.at[pl.ds(0, k * 8)],
                        t0_hbm.at[pl.ds(base * 8, k * 8)])

    def loop_body(i, carry):
        idx = w + i * NW

        @pl.when(idx < N_FULL)
        def _():
            do_chunk(idx * CHUNK, CHUNK)

        return carry

    lax.fori_loop(0, NODE_ITERS, loop_body, 0)

    @pl.when(w == N_FULL % NW)
    def _():
        do_chunk(N_FULL * CHUNK, N_TAIL)


def _build_t0(nt_flat, labels, emb1):
    fn = pl.kernel(
        _build_t0_body,
        out_type=jax.ShapeDtypeStruct((N * 8,), jnp.float32),
        mesh=_mesh(),
        scratch_types=[
            pltpu.VMEM((NUM_TOK,), jnp.float32),
            pltpu.VMEM((CHUNK * 6,), jnp.float32),
            pltpu.VMEM((CHUNK,), jnp.int32),
            pltpu.VMEM((CHUNK * 8,), jnp.float32),
        ],
        compiler_params=pltpu.CompilerParams(needs_layout_passes=False, use_tc_tiling_on_sc=False),
    )
    return fn(nt_flat, labels, emb1)


def _edge0_body(src_hbm, dst_hbm, lab_hbm, etT_hbm, emb_hbm, t0_hbm,
                z8_hbm, z1_hbm,
                outA_hbm, outB_hbm, outC_hbm,
                emb_v, srcb, dstb, labb, etb0, etb1, etb2, etb3, rowsv, ecb,
                accA, accB0, accB1, accB2, accB3, accC, sem):
    c = lax.axis_index("c")
    s = lax.axis_index("s")
    w = s * NC + c
    pltpu.sync_copy(emb_hbm, emb_v)

    r0 = s * NPS
    pltpu.sync_copy(z8_hbm.at[pl.ds(r0, NPS), :], accA.at[pl.ds(r0, NPS), :])
    for accB_c in (accB0, accB1, accB2, accB3):
        pltpu.sync_copy(z1_hbm.at[pl.ds(r0, NPS)], accB_c.at[pl.ds(r0, NPS)])
    pltpu.sync_copy(z1_hbm.at[pl.ds(r0, NPS)], accC.at[pl.ds(r0, NPS)])
    plsc.subcore_barrier()

    etbufs = (etb0, etb1, etb2, etb3)
    accBs = (accB0, accB1, accB2, accB3)

    def echunk(off):
        pltpu.sync_copy(src_hbm.at[pl.ds(off, CHUNK)], srcb)
        pltpu.sync_copy(dst_hbm.at[pl.ds(off, CHUNK)], dstb)
        pltpu.sync_copy(lab_hbm.at[pl.ds(off, CHUNK)], labb)
        for c_ in range(4):
            pltpu.sync_copy(etT_hbm.at[pl.ds(c_ * E + off, CHUNK)], etbufs[c_])
        pltpu.async_copy(t0_hbm.at[srcb], rowsv, sem).wait()
        for g in range(CHUNK // L):
            labs = labb[pl.ds(g * L, L)]
            ecb[pl.ds(g * L, L)] = plsc.load_gather(emb_v, [labs])
        pltpu.sync_copy(rowsv, accA.at[dstb], add=True)
        for c_ in range(4):
            pltpu.sync_copy(etbufs[c_], accBs[c_].at[dstb], add=True)
        pltpu.sync_copy(ecb, accC.at[dstb], add=True)

    def eloop(i, carry):
        idx = w + i * NW

        @pl.when(idx < E_CHUNKS)
        def _():
            echunk(idx * CHUNK)

        return carry

    lax.fori_loop(0, E_ITERS, eloop, 0)

    plsc.subcore_barrier()
    pltpu.sync_copy(accA.at[pl.ds(r0, NPS), :], outA_hbm.at[c, pl.ds(r0, NPS), :])
    for c_ in range(4):
        pltpu.sync_copy(accBs[c_].at[pl.ds(r0, NPS)],
                        outB_hbm.at[c_, c, pl.ds(r0, NPS)])
    pltpu.sync_copy(accC.at[pl.ds(r0, NPS)], outC_hbm.at[c, pl.ds(r0, NPS)])


def _edge_pass0(src, dst, elab, etT_flat, emb1, t0_2d, z8, z1):
    fn = pl.kernel(
        _edge0_body,
        out_type=(jax.ShapeDtypeStruct((2, N_ACC, 8), jnp.float32),
                  jax.ShapeDtypeStruct((4, 2, N_ACC), jnp.float32),
                  jax.ShapeDtypeStruct((2, N_ACC), jnp.float32)),
        mesh=_mesh(),
        scratch_types=[
            pltpu.VMEM((NUM_TOK,), jnp.float32),
            pltpu.VMEM((CHUNK,), jnp.int32),
            pltpu.VMEM((CHUNK,), jnp.int32),
            pltpu.VMEM((CHUNK,), jnp.int32),
            pltpu.VMEM((CHUNK,), jnp.float32),
            pltpu.VMEM((CHUNK,), jnp.float32),
            pltpu.VMEM((CHUNK,), jnp.float32),
            pltpu.VMEM((CHUNK,), jnp.float32),
            pltpu.VMEM((CHUNK, 8), jnp.float32),
            pltpu.VMEM((CHUNK,), jnp.float32),
            pltpu.VMEM_SHARED((N_ACC, 8), jnp.float32),
            pltpu.VMEM_SHARED((N_ACC,), jnp.float32),
            pltpu.VMEM_SHARED((N_ACC,), jnp.float32),
            pltpu.VMEM_SHARED((N_ACC,), jnp.float32),
            pltpu.VMEM_SHARED((N_ACC,), jnp.float32),
            pltpu.VMEM_SHARED((N_ACC,), jnp.float32),
            pltpu.SemaphoreType.DMA,
        ],
        compiler_params=pltpu.CompilerParams(needs_layout_passes=False, use_tc_tiling_on_sc=False),
    )
    return fn(src, dst, elab, etT_flat, emb1, t0_2d, z8, z1)


def _edge1_body(src_hbm, dst_hbm, h1_hbm, z16_hbm, out_hbm,
                srcb, dstb, rowsv, acc1, sem):
    c = lax.axis_index("c")
    s = lax.axis_index("s")
    w = s * NC + c

    r0 = s * NPS
    pltpu.sync_copy(z16_hbm.at[pl.ds(r0, NPS), :], acc1.at[pl.ds(r0, NPS), :])
    plsc.subcore_barrier()

    def echunk(off):
        pltpu.sync_copy(src_hbm.at[pl.ds(off, CHUNK)], srcb)
        pltpu.sync_copy(dst_hbm.at[pl.ds(off, CHUNK)], dstb)
        pltpu.async_copy(h1_hbm.at[srcb], rowsv, sem).wait()
        pltpu.sync_copy(rowsv, acc1.at[dstb], add=True)

    def eloop(i, carry):
        idx = w + i * NW

        @pl.when(idx < E_CHUNKS)
        def _():
            echunk(idx * CHUNK)

        return carry

    lax.fori_loop(0, E_ITERS, eloop, 0)

    plsc.subcore_barrier()
    pltpu.sync_copy(acc1.at[pl.ds(r0, NPS), :], out_hbm.at[c, pl.ds(r0, NPS), :])


def _edge_pass1(src, dst, h1, z16):
    fn = pl.kernel(
        _edge1_body,
        out_type=jax.ShapeDtypeStruct((2, N_ACC, 16), jnp.float32),
        mesh=_mesh(),
        scratch_types=[
            pltpu.VMEM((CHUNK,), jnp.int32),
            pltpu.VMEM((CHUNK,), jnp.int32),
            pltpu.VMEM((CHUNK, 16), jnp.float32),
            pltpu.VMEM_SHARED((N_ACC, 16), jnp.float32),
            pltpu.SemaphoreType.DMA,
        ],
        compiler_params=pltpu.CompilerParams(needs_layout_passes=False, use_tc_tiling_on_sc=False),
    )
    return fn(src, dst, h1, z16)


BN = 4000


def _tca_body(t0_ref, aA_ref, aB_ref, aC_ref, wa_ref, wb_ref, wc_ref, we_ref,
              b_ref, out_ref):
    x0 = t0_ref[...]
    a = aA_ref[0] + aA_ref[1]
    b4 = aB_ref[0] + aB_ref[1]
    cc = aC_ref[0] + aC_ref[1]
    h = (jnp.dot(x0, wa_ref[...], preferred_element_type=jnp.float32)
         + jnp.dot(a, wb_ref[...], preferred_element_type=jnp.float32)
         + jnp.dot(b4, wc_ref[...], preferred_element_type=jnp.float32)
         + cc * we_ref[...]
         + b_ref[...])
    out_ref[...] = jnp.maximum(h, 0.0)


def _tca(t0, accA, accB, accC, Wa, Wb, Wc, we, b0r):
    grid = (N // BN,)
    return pl.pallas_call(
        _tca_body,
        grid=grid,
        in_specs=[
            pl.BlockSpec((BN, 8), lambda i: (i, 0)),
            pl.BlockSpec((2, BN, 8), lambda i: (0, i, 0)),
            pl.BlockSpec((2, BN, 4), lambda i: (0, i, 0)),
            pl.BlockSpec((2, BN, 1), lambda i: (0, i, 0)),
            pl.BlockSpec((8, H), lambda i: (0, 0)),
            pl.BlockSpec((8, H), lambda i: (0, 0)),
            pl.BlockSpec((4, H), lambda i: (0, 0)),
            pl.BlockSpec((1, H), lambda i: (0, 0)),
            pl.BlockSpec((1, H), lambda i: (0, 0)),
        ],
        out_specs=pl.BlockSpec((BN, H), lambda i: (i, 0)),
        out_shape=jax.ShapeDtypeStruct((N, H), jnp.float32),
    )(t0, accA, accB, accC, Wa, Wb, Wc, we, b0r)


def _tcb_body(h1_ref, a1_ref, aB_ref, aC_ref, vh_ref, vs_ref, vb_ref, ve_ref,
              b1_ref, fcw_ref, fcb_ref, out_ref, acc_ref):
    i = pl.program_id(0)

    @pl.when(i == 0)
    def _():
        acc_ref[...] = jnp.zeros_like(acc_ref)

    h1 = h1_ref[...]
    a1 = a1_ref[0] + a1_ref[1]
    aB4 = aB_ref[0] + aB_ref[1]
    aC = aC_ref[0] + aC_ref[1]
    h2 = (jnp.dot(h1, vh_ref[...], preferred_element_type=jnp.float32)
          + jnp.dot(a1, vs_ref[...], preferred_element_type=jnp.float32)
          + jnp.dot(aB4, vb_ref[...], preferred_element_type=jnp.float32)
          + aC * ve_ref[...]
          + b1_ref[...])
    h2 = jnp.maximum(h2, 0.0)
    acc_ref[...] += jnp.sum(h2, axis=0, keepdims=True)

    @pl.when(i == pl.num_programs(0) - 1)
    def _():
        out_ref[...] = (jnp.dot(acc_ref[...], fcw_ref[...],
                                preferred_element_type=jnp.float32)
                        + fcb_ref[...])


def _tcb(h1, acc1, accB, accC, Vh, Vs, Vb, ve, b1r, fcw, fcbr):
    C = fcw.shape[1]
    grid = (N // BN,)
    return pl.pallas_call(
        _tcb_body,
        grid=grid,
        in_specs=[
            pl.BlockSpec((BN, H), lambda i: (i, 0)),
            pl.BlockSpec((2, BN, H), lambda i: (0, i, 0)),
            pl.BlockSpec((2, BN, 4), lambda i: (0, i, 0)),
            pl.BlockSpec((2, BN, 1), lambda i: (0, i, 0)),
            pl.BlockSpec((H, H), lambda i: (0, 0)),
            pl.BlockSpec((H, H), lambda i: (0, 0)),
            pl.BlockSpec((4, H), lambda i: (0, 0)),
            pl.BlockSpec((1, H), lambda i: (0, 0)),
            pl.BlockSpec((1, H), lambda i: (0, 0)),
            pl.BlockSpec((H, C), lambda i: (0, 0)),
            pl.BlockSpec((1, C), lambda i: (0, 0)),
        ],
        out_specs=pl.BlockSpec((1, C), lambda i: (0, 0)),
        out_shape=jax.ShapeDtypeStruct((1, C), jnp.float32),
        scratch_shapes=[pltpu.VMEM((1, H), jnp.float32)],
    )(h1, acc1, accB, accC, Vh, Vs, Vb, ve, b1r, fcw, fcbr)


def kernel(node_types, node_labels, edge_types, edge_labels, edge_index, emb,
           W_node0, W_src0, W_edge0, b0, W_node1, W_src1, W_edge1, b1,
           fc_w, fc_b):
    src = edge_index[0].astype(jnp.int32)
    dst = edge_index[1].astype(jnp.int32)
    elab = edge_labels.astype(jnp.int32)
    nlab = node_labels.astype(jnp.int32)
    emb1 = emb.reshape(-1)
    nt_flat = node_types.reshape(-1)

    t0_flat = _build_t0(nt_flat, nlab, emb1)
    t0 = t0_flat.reshape(N, 8)

    z8 = jnp.zeros((N_ACC, 8), jnp.float32)
    z1 = jnp.zeros((N_ACC,), jnp.float32)
    z16 = jnp.zeros((N_ACC, 16), jnp.float32)

    etT_flat = edge_types.T.reshape(-1)
    accA, accB, accC = _edge_pass0(src, dst, elab, etT_flat, emb1, t0, z8, z1)
    accC = accC.reshape(2, N_ACC, 1)
    accB = accB.transpose(1, 2, 0)

    zrow = jnp.zeros((1, H), jnp.float32)
    Wa = jnp.concatenate([W_node0, zrow], axis=0)
    Wb = jnp.concatenate([W_src0, zrow], axis=0)
    Wc = W_edge0[0:4]
    we0 = W_edge0[4:5]
    h1 = _tca(t0, accA, accB, accC, Wa, Wb, Wc, we0, b0.reshape(1, H))

    acc1 = _edge_pass1(src, dst, h1, z16)

    Vb = W_edge1[0:4]
    we1 = W_edge1[4:5]
    out = _tcb(h1, acc1, accB, accC, W_node1, W_src1, Vb, we1,
               b1.reshape(1, H), fc_w, fc_b.reshape(1, -1))
    return out
